# baseline (speedup 1.0000x reference)
import functools

import jax
import jax.numpy as jnp
from jax import lax
from jax.experimental import pallas as pl
from jax.experimental.pallas import tpu as pltpu
from jax.experimental.pallas import tpu_sc as plsc

N = 10000
HD = 64
E = 320000
NDIM = 128

NSUB = 16
CH = 100
EPT = E // NSUB
CPT = EPT // CH
ROWS_A = 640
ROWS_B = N - 15 * ROWS_A
D0 = 80
NB = 4


def _sc_scatter():
  mesh = plsc.VectorSubcoreMesh(core_axis_name="c", subcore_axis_name="s")

  def body(tf, tb, em, zrows, gf, gb, idxg, idxs, rows, g_sh, semg, sems):
    c = lax.axis_index("c")
    s = lax.axis_index("s")

    @pl.when(c == 0)
    def _():
      pltpu.sync_copy(em.at[0, pl.ds(s * CPT, CPT)], idxg)
      pltpu.sync_copy(em.at[1, pl.ds(s * CPT, CPT)], idxs)

    @pl.when(c == 1)
    def _():
      pltpu.sync_copy(em.at[1, pl.ds(s * CPT, CPT)], idxg)
      pltpu.sync_copy(em.at[0, pl.ds(s * CPT, CPT)], idxs)

    @pl.when(s < 15)
    def _():
      pltpu.sync_copy(zrows, g_sh.at[pl.ds(s * ROWS_A, ROWS_A)])

    @pl.when(s == 15)
    def _():
      pltpu.sync_copy(zrows.at[pl.ds(0, ROWS_B)],
                      g_sh.at[pl.ds(15 * ROWS_A, ROWS_B)])

    plsc.subcore_barrier()

    def run_dir(table, out):
      for b in range(NB):
        pltpu.async_copy(table.at[idxg.at[b]], rows.at[b], semg)

      def group(i, carry):
        for b in range(NB):
          j = i * NB + b
          pltpu.make_async_copy(table.at[idxg.at[j]], rows.at[b], semg).wait()
          pltpu.async_copy(rows.at[b], g_sh.at[idxs.at[j]], sems, add=True)
          pltpu.make_async_copy(rows.at[b], g_sh.at[idxs.at[j]], sems).wait()

          @pl.when(j < CPT - NB)
          def _():
            pltpu.async_copy(table.at[idxg.at[j + NB]], rows.at[b], semg)

        return carry

      lax.fori_loop(0, CPT // NB, group, 0)
      plsc.subcore_barrier()

      @pl.when(s < 15)
      def _():
        pltpu.sync_copy(g_sh.at[pl.ds(s * ROWS_A, ROWS_A)],
                        out.at[pl.ds(s * ROWS_A, ROWS_A)])

      @pl.when(s == 15)
      def _():
        pltpu.sync_copy(g_sh.at[pl.ds(15 * ROWS_A, ROWS_B)],
                        out.at[pl.ds(15 * ROWS_A, ROWS_B)])

    @pl.when(c == 0)
    def _():
      run_dir(tf, gf)

    @pl.when(c == 1)
    def _():
      run_dir(tb, gb)

  return pl.kernel(
      body,
      out_type=(
          jax.ShapeDtypeStruct((N, D0), jnp.float32),
          jax.ShapeDtypeStruct((N, D0), jnp.float32),
      ),
      mesh=mesh,
      compiler_params=pltpu.CompilerParams(use_tc_tiling_on_sc=False),
      scratch_types=[
          pltpu.VMEM((CPT, CH), jnp.int32),
          pltpu.VMEM((CPT, CH), jnp.int32),
          pltpu.VMEM((NB, CH, D0), jnp.float32),
          pltpu.VMEM_SHARED((N, D0), jnp.float32),
          pltpu.SemaphoreType.DMA,
          pltpu.SemaphoreType.DMA,
      ],
  )


BN = 2000


def _gru_dir(l, h_ref, g_ref, wm, wih, whh, bm, bih, bhh):
  f32 = jnp.float32
  bf = jnp.bfloat16
  h = h_ref[...]
  g80 = g_ref[...]
  g = g80[:, :HD]
  deg = g80[:, HD:HD + 1]
  wm_ = wm[...].reshape(NDIM, NDIM)
  w1r = wm_[:, :HD].astype(bf).astype(f32)
  w2 = wm_[:, HD:].astype(bf)
  h16 = h.astype(bf)
  ddot = lambda a, b, **kw: lax.dot_general(
      a, b, (((1,), (1,)), ((), ())), preferred_element_type=f32, **kw)
  b_part = ddot(h16, w2) + bm[...][l:l + 1]
  aggr = ddot(g, w1r, precision=lax.Precision.HIGHEST) + deg * b_part
  gi = ddot(aggr.astype(bf), wih[...].reshape(3 * HD, NDIM).astype(bf))
  gi = gi + bih[...][l:l + 1]
  gh = ddot(h16, whh[...].reshape(3 * HD, HD).astype(bf)) + bhh[...][l:l + 1]
  r = jax.nn.sigmoid(gi[:, :HD] + gh[:, :HD])
  z = jax.nn.sigmoid(gi[:, HD:2 * HD] + gh[:, HD:2 * HD])
  n = jnp.tanh(gi[:, 2 * HD:] + r * gh[:, 2 * HD:])
  return (1.0 - z) * n + z * h


def _tc_body(final, l, hf, hb, gf, gb,
             wmf, wihf, whhf, bmf, bihf, bhhf,
             wmb, wihb, whhb, bmb, bihb, bhhb,
             *outs):
  hf2 = _gru_dir(l, hf, gf, wmf, wihf, whhf, bmf, bihf, bhhf)
  hb2 = _gru_dir(l, hb, gb, wmb, wihb, whhb, bmb, bihb, bhhb)
  if final:
    outs[0][...] = jnp.concatenate([hf2, hb2], axis=1)
  else:
    outs[0][...] = hf2
    outs[1][...] = hb2
    rows = hf2.shape[0]
    ones = jnp.ones((rows, 1), jnp.float32)
    zpad = jnp.zeros((rows, D0 - HD - 1), jnp.float32)
    rnd16 = lambda x: x.astype(jnp.bfloat16).astype(jnp.float32)
    outs[2][...] = jnp.concatenate([rnd16(hf2), ones, zpad], axis=1)
    outs[3][...] = jnp.concatenate([rnd16(hb2), ones, zpad], axis=1)


def _tc_layer(final, l):
  data = lambda cols: pl.BlockSpec((BN, cols), lambda i: (i, 0))
  wspec = lambda a, b: pl.BlockSpec((1, a, b), lambda i: (l, 0, 0))
  bspec = lambda a: pl.BlockSpec((2, a), lambda i: (0, 0))
  wspecs = [wspec(NDIM, NDIM), wspec(3 * HD, NDIM), wspec(3 * HD, HD),
            bspec(NDIM), bspec(3 * HD), bspec(3 * HD)]
  if final:
    out_specs = [data(NDIM)]
    out_shape = [jax.ShapeDtypeStruct((N, NDIM), jnp.float32)]
  else:
    out_specs = [data(HD), data(HD), data(D0), data(D0)]
    out_shape = [jax.ShapeDtypeStruct((N, HD), jnp.float32)] * 2 + [
        jax.ShapeDtypeStruct((N, D0), jnp.float32)] * 2
  return pl.pallas_call(
      functools.partial(_tc_body, final, l),
      grid=(N // BN,),
      in_specs=[data(HD)] * 2 + [data(D0)] * 2 + wspecs + wspecs,
      out_specs=out_specs,
      out_shape=out_shape,
  )


def kernel(h, edge_index, Wmsg_f, bmsg_f, Wih_f, Whh_f, bih_f, bhh_f,
           Wmsg_b, bmsg_b, Wih_b, Whh_b, bih_b, bhh_b):
  hf = h[:, :HD]
  hb = h[:, HD:]
  em = edge_index.reshape(2, E // CH, CH)

  rnd = lambda x: x.astype(jnp.bfloat16).astype(jnp.float32)
  pad = jnp.concatenate(
      [jnp.ones((N, 1), jnp.float32),
       jnp.zeros((N, D0 - HD - 1), jnp.float32)], axis=1)
  tf0 = jnp.concatenate([rnd(hf), pad], axis=1)
  tb0 = jnp.concatenate([rnd(hb), pad], axis=1)
  zrows = jnp.zeros((ROWS_A, D0), jnp.float32)

  sc = _sc_scatter()
  gf0, gb0 = sc(tf0, tb0, em, zrows)
  wf = (Wmsg_f, Wih_f, Whh_f, bmsg_f, bih_f, bhh_f)
  wb = (Wmsg_b, Wih_b, Whh_b, bmsg_b, bih_b, bhh_b)
  hf1, hb1, tf1, tb1 = _tc_layer(False, 0)(hf, hb, gf0, gb0, *wf, *wb)

  gf1, gb1 = sc(tf1, tb1, em, zrows)
  (out,) = _tc_layer(True, 1)(hf1, hb1, gf1, gb1, *wf, *wb)
  return out


# speedup vs baseline: 1.0955x; 1.0955x over previous
import functools

import jax
import jax.numpy as jnp
from jax import lax
from jax.experimental import pallas as pl
from jax.experimental.pallas import tpu as pltpu
from jax.experimental.pallas import tpu_sc as plsc

N = 10000
HD = 64
E = 320000
NDIM = 128

NSUB = 16
CH = 100
EPT = E // NSUB
CPT = EPT // CH
ROWS_A = 640
ROWS_B = N - 15 * ROWS_A
D0 = 80
NB = 4


def _sc_scatter(D):
  mesh = plsc.VectorSubcoreMesh(core_axis_name="c", subcore_axis_name="s")

  def body(tf, tb, em, zrows, g_out, idxg, idxs, rows, g_sh, semg, sems):
    c = lax.axis_index("c")
    s = lax.axis_index("s")

    @pl.when(c == 0)
    def _():
      pltpu.sync_copy(em.at[0, pl.ds(s * CPT, CPT)], idxg)
      pltpu.sync_copy(em.at[1, pl.ds(s * CPT, CPT)], idxs)

    @pl.when(c == 1)
    def _():
      pltpu.sync_copy(em.at[1, pl.ds(s * CPT, CPT)], idxg)
      pltpu.sync_copy(em.at[0, pl.ds(s * CPT, CPT)], idxs)

    @pl.when(s < 15)
    def _():
      pltpu.sync_copy(zrows, g_sh.at[pl.ds(s * ROWS_A, ROWS_A)])

    @pl.when(s == 15)
    def _():
      pltpu.sync_copy(zrows.at[pl.ds(0, ROWS_B)],
                      g_sh.at[pl.ds(15 * ROWS_A, ROWS_B)])

    plsc.subcore_barrier()

    def run_dir(table, cidx):
      for b in range(NB):
        pltpu.async_copy(table.at[idxg.at[b]], rows.at[b], semg)

      def group(i, carry):
        for b in range(NB):
          j = i * NB + b
          pltpu.make_async_copy(table.at[idxg.at[j]], rows.at[b], semg).wait()
          pltpu.async_copy(rows.at[b], g_sh.at[idxs.at[j]], sems, add=True)
          pltpu.make_async_copy(rows.at[b], g_sh.at[idxs.at[j]], sems).wait()

          @pl.when(j < CPT - NB)
          def _():
            pltpu.async_copy(table.at[idxg.at[j + NB]], rows.at[b], semg)

        return carry

      lax.fori_loop(0, CPT // NB, group, 0)
      plsc.subcore_barrier()

      @pl.when(s < 15)
      def _():
        pltpu.sync_copy(g_sh.at[pl.ds(s * ROWS_A, ROWS_A)],
                        g_out.at[cidx, pl.ds(s * ROWS_A, ROWS_A)])

      @pl.when(s == 15)
      def _():
        pltpu.sync_copy(g_sh.at[pl.ds(15 * ROWS_A, ROWS_B)],
                        g_out.at[cidx, pl.ds(15 * ROWS_A, ROWS_B)])

    @pl.when(c == 0)
    def _():
      run_dir(tf, 0)

    @pl.when(c == 1)
    def _():
      run_dir(tb, 1)

  return pl.kernel(
      body,
      out_type=jax.ShapeDtypeStruct((2, N, D), jnp.float32),
      mesh=mesh,
      compiler_params=pltpu.CompilerParams(use_tc_tiling_on_sc=False),
      scratch_types=[
          pltpu.VMEM((CPT, CH), jnp.int32),
          pltpu.VMEM((CPT, CH), jnp.int32),
          pltpu.VMEM((NB, CH, D), jnp.float32),
          pltpu.VMEM_SHARED((N, D), jnp.float32),
          pltpu.SemaphoreType.DMA,
          pltpu.SemaphoreType.DMA,
      ],
  )


BN = 2000


def _gru_dir(l, h, g, deg, wm, wih, whh, bm, bih, bhh):
  f32 = jnp.float32
  bf = jnp.bfloat16
  wm_ = wm[...].reshape(NDIM, NDIM)
  w1r = wm_[:, :HD].astype(bf).astype(f32)
  w2 = wm_[:, HD:].astype(bf)
  h16 = h.astype(bf)
  ddot = lambda a, b, **kw: lax.dot_general(
      a, b, (((1,), (1,)), ((), ())), preferred_element_type=f32, **kw)
  b_part = ddot(h16, w2) + bm[...][l:l + 1]
  aggr = ddot(g, w1r, precision=lax.Precision.HIGHEST) + deg * b_part
  gi = ddot(aggr.astype(bf), wih[...].reshape(3 * HD, NDIM).astype(bf))
  gi = gi + bih[...][l:l + 1]
  gh = ddot(h16, whh[...].reshape(3 * HD, HD).astype(bf)) + bhh[...][l:l + 1]
  r = jax.nn.sigmoid(gi[:, :HD] + gh[:, :HD])
  z = jax.nn.sigmoid(gi[:, HD:2 * HD] + gh[:, HD:2 * HD])
  n = jnp.tanh(gi[:, 2 * HD:] + r * gh[:, 2 * HD:])
  return (1.0 - z) * n + z * h


def _tc_body0(hin, gf, gb,
              wmf, wihf, whhf, bmf, bihf, bhhf,
              wmb, wihb, whhb, bmb, bihb, bhhb,
              hdf, hdb, tf1, tb1):
  h128 = hin[...]
  rnd16 = lambda x: x.astype(jnp.bfloat16).astype(jnp.float32)
  zpad = jnp.zeros((h128.shape[0], NDIM - HD - 1), jnp.float32)
  for (gref, c0, hd_out, t_out, w) in (
      (gf, 0, hdf, tf1, (wmf, wihf, whhf, bmf, bihf, bhhf)),
      (gb, HD, hdb, tb1, (wmb, wihb, whhb, bmb, bihb, bhhb))):
    g80 = gref[...].reshape(-1, D0)
    h2 = _gru_dir(0, h128[:, c0:c0 + HD], g80[:, :HD], g80[:, HD:HD + 1], *w)
    hd_out[...] = jnp.concatenate([h2, g80[:, HD:HD + 1], zpad], axis=1)
    t_out[...] = rnd16(h2)


def _tc_body1(hdf, hdb, gf, gb,
              wmf, wihf, whhf, bmf, bihf, bhhf,
              wmb, wihb, whhb, bmb, bihb, bhhb,
              out):
  res = []
  for (hd, gref, w) in (
      (hdf, gf, (wmf, wihf, whhf, bmf, bihf, bhhf)),
      (hdb, gb, (wmb, wihb, whhb, bmb, bihb, bhhb))):
    h128 = hd[...]
    g = gref[...].reshape(-1, HD)
    res.append(_gru_dir(1, h128[:, :HD], g, h128[:, HD:HD + 1], *w))
  out[...] = jnp.concatenate(res, axis=1)


def _wspecs():
  wspec = lambda l, a, b: pl.BlockSpec((1, a, b), lambda i: (l, 0, 0))
  bspec = lambda a: pl.BlockSpec((2, a), lambda i: (0, 0))
  return lambda l: [wspec(l, NDIM, NDIM), wspec(l, 3 * HD, NDIM),
                    wspec(l, 3 * HD, HD), bspec(NDIM), bspec(3 * HD),
                    bspec(3 * HD)]


def _tc_layer0():
  data = lambda cols: pl.BlockSpec((BN, cols), lambda i: (i, 0))
  gview = lambda d: pl.BlockSpec((1, BN, D0), lambda i: (d, i, 0))
  w = _wspecs()(0)
  return pl.pallas_call(
      _tc_body0,
      grid=(N // BN,),
      in_specs=[data(NDIM), gview(0), gview(1)] + w + w,
      out_specs=[data(NDIM), data(NDIM), data(HD), data(HD)],
      out_shape=[jax.ShapeDtypeStruct((N, NDIM), jnp.float32)] * 2 + [
          jax.ShapeDtypeStruct((N, HD), jnp.float32)] * 2,
  )


def _tc_layer1():
  data = lambda cols: pl.BlockSpec((BN, cols), lambda i: (i, 0))
  gview = lambda d: pl.BlockSpec((1, BN, HD), lambda i: (d, i, 0))
  w = _wspecs()(1)
  return pl.pallas_call(
      _tc_body1,
      grid=(N // BN,),
      in_specs=[data(NDIM), data(NDIM), gview(0), gview(1)] + w + w,
      out_specs=[data(NDIM)],
      out_shape=[jax.ShapeDtypeStruct((N, NDIM), jnp.float32)],
  )


def kernel(h, edge_index, Wmsg_f, bmsg_f, Wih_f, Whh_f, bih_f, bhh_f,
           Wmsg_b, bmsg_b, Wih_b, Whh_b, bih_b, bhh_b):
  em = edge_index.reshape(2, E // CH, CH)

  rnd = lambda x: x.astype(jnp.bfloat16).astype(jnp.float32)
  pad = jnp.concatenate(
      [jnp.ones((N, 1), jnp.float32),
       jnp.zeros((N, D0 - HD - 1), jnp.float32)], axis=1)
  tf0 = jnp.concatenate([rnd(h[:, :HD]), pad], axis=1)
  tb0 = jnp.concatenate([rnd(h[:, HD:]), pad], axis=1)

  g0 = _sc_scatter(D0)(tf0, tb0, em, jnp.zeros((ROWS_A, D0), jnp.float32))
  wf = (Wmsg_f, Wih_f, Whh_f, bmsg_f, bih_f, bhh_f)
  wb = (Wmsg_b, Wih_b, Whh_b, bmsg_b, bih_b, bhh_b)
  hdf, hdb, tf1, tb1 = _tc_layer0()(h, g0, g0, *wf, *wb)

  g1 = _sc_scatter(HD)(tf1, tb1, em, jnp.zeros((ROWS_A, HD), jnp.float32))
  (out,) = _tc_layer1()(hdf, hdb, g1, g1, *wf, *wb)
  return out


# speedup vs baseline: 1.1111x; 1.0143x over previous
import functools

import jax
import jax.numpy as jnp
from jax import lax
from jax.experimental import pallas as pl
from jax.experimental.pallas import tpu as pltpu
from jax.experimental.pallas import tpu_sc as plsc

N = 10000
HD = 64
E = 320000
NDIM = 128

NSUB = 16
CH = 100
EPT = E // NSUB
CPT = EPT // CH
ROWS_A = 640
ROWS_B = N - 15 * ROWS_A
D0 = 72
NB = 4


def _sc_scatter(D):
  mesh = plsc.VectorSubcoreMesh(core_axis_name="c", subcore_axis_name="s")

  def body(tf, tb, em, zrows, g_out, idxg, idxs, rows, g_sh, semg, sems):
    c = lax.axis_index("c")
    s = lax.axis_index("s")

    @pl.when(c == 0)
    def _():
      pltpu.sync_copy(em.at[0, pl.ds(s * CPT, CPT)], idxg)
      pltpu.sync_copy(em.at[1, pl.ds(s * CPT, CPT)], idxs)

    @pl.when(c == 1)
    def _():
      pltpu.sync_copy(em.at[1, pl.ds(s * CPT, CPT)], idxg)
      pltpu.sync_copy(em.at[0, pl.ds(s * CPT, CPT)], idxs)

    @pl.when(s < 15)
    def _():
      pltpu.sync_copy(zrows, g_sh.at[pl.ds(s * ROWS_A, ROWS_A)])

    @pl.when(s == 15)
    def _():
      pltpu.sync_copy(zrows.at[pl.ds(0, ROWS_B)],
                      g_sh.at[pl.ds(15 * ROWS_A, ROWS_B)])

    plsc.subcore_barrier()

    def run_dir(table, cidx):
      for b in range(NB):
        pltpu.async_copy(table.at[idxg.at[b]], rows.at[b], semg)

      def group(i, carry):
        for b in range(NB):
          j = i * NB + b
          pltpu.make_async_copy(table.at[idxg.at[j]], rows.at[b], semg).wait()
          pltpu.async_copy(rows.at[b], g_sh.at[idxs.at[j]], sems, add=True)
          pltpu.make_async_copy(rows.at[b], g_sh.at[idxs.at[j]], sems).wait()

          @pl.when(j < CPT - NB)
          def _():
            pltpu.async_copy(table.at[idxg.at[j + NB]], rows.at[b], semg)

        return carry

      lax.fori_loop(0, CPT // NB, group, 0)
      plsc.subcore_barrier()

      @pl.when(s < 15)
      def _():
        pltpu.sync_copy(g_sh.at[pl.ds(s * ROWS_A, ROWS_A)],
                        g_out.at[cidx, pl.ds(s * ROWS_A, ROWS_A)])

      @pl.when(s == 15)
      def _():
        pltpu.sync_copy(g_sh.at[pl.ds(15 * ROWS_A, ROWS_B)],
                        g_out.at[cidx, pl.ds(15 * ROWS_A, ROWS_B)])

    @pl.when(c == 0)
    def _():
      run_dir(tf, 0)

    @pl.when(c == 1)
    def _():
      run_dir(tb, 1)

  return pl.kernel(
      body,
      out_type=jax.ShapeDtypeStruct((2, N, D), jnp.float32),
      mesh=mesh,
      compiler_params=pltpu.CompilerParams(use_tc_tiling_on_sc=False),
      scratch_types=[
          pltpu.VMEM((CPT, CH), jnp.int32),
          pltpu.VMEM((CPT, CH), jnp.int32),
          pltpu.VMEM((NB, CH, D), jnp.float32),
          pltpu.VMEM_SHARED((N, D), jnp.float32),
          pltpu.SemaphoreType.DMA,
          pltpu.SemaphoreType.DMA,
      ],
  )


BN = 2000


def _gru_dir(l, h, g, deg, wm, wih, whh, bm, bih, bhh):
  f32 = jnp.float32
  bf = jnp.bfloat16
  wm_ = wm[...].reshape(NDIM, NDIM)
  w1r = wm_[:, :HD].astype(bf).astype(f32)
  w2 = wm_[:, HD:].astype(bf)
  h16 = h.astype(bf)
  ddot = lambda a, b, **kw: lax.dot_general(
      a, b, (((1,), (1,)), ((), ())), preferred_element_type=f32, **kw)
  b_part = ddot(h16, w2) + bm[...][l:l + 1]
  aggr = ddot(g, w1r, precision=lax.Precision.HIGHEST) + deg * b_part
  gi = ddot(aggr.astype(bf), wih[...].reshape(3 * HD, NDIM).astype(bf))
  gi = gi + bih[...][l:l + 1]
  gh = ddot(h16, whh[...].reshape(3 * HD, HD).astype(bf)) + bhh[...][l:l + 1]
  r = jax.nn.sigmoid(gi[:, :HD] + gh[:, :HD])
  z = jax.nn.sigmoid(gi[:, HD:2 * HD] + gh[:, HD:2 * HD])
  n = jnp.tanh(gi[:, 2 * HD:] + r * gh[:, 2 * HD:])
  return (1.0 - z) * n + z * h


def _tc_body0(hin, gf, gb,
              wmf, wihf, whhf, bmf, bihf, bhhf,
              wmb, wihb, whhb, bmb, bihb, bhhb,
              hdf, hdb, tf1, tb1):
  h128 = hin[...]
  rnd16 = lambda x: x.astype(jnp.bfloat16).astype(jnp.float32)
  zpad = jnp.zeros((h128.shape[0], NDIM - HD - 1), jnp.float32)
  for (gref, c0, hd_out, t_out, w) in (
      (gf, 0, hdf, tf1, (wmf, wihf, whhf, bmf, bihf, bhhf)),
      (gb, HD, hdb, tb1, (wmb, wihb, whhb, bmb, bihb, bhhb))):
    g80 = gref[...].reshape(-1, D0)
    h2 = _gru_dir(0, h128[:, c0:c0 + HD], g80[:, :HD], g80[:, HD:HD + 1], *w)
    hd_out[...] = jnp.concatenate([h2, g80[:, HD:HD + 1], zpad], axis=1)
    t_out[...] = rnd16(h2)


def _tc_body1(hdf, hdb, gf, gb,
              wmf, wihf, whhf, bmf, bihf, bhhf,
              wmb, wihb, whhb, bmb, bihb, bhhb,
              out):
  res = []
  for (hd, gref, w) in (
      (hdf, gf, (wmf, wihf, whhf, bmf, bihf, bhhf)),
      (hdb, gb, (wmb, wihb, whhb, bmb, bihb, bhhb))):
    h128 = hd[...]
    g = gref[...].reshape(-1, HD)
    res.append(_gru_dir(1, h128[:, :HD], g, h128[:, HD:HD + 1], *w))
  out[...] = jnp.concatenate(res, axis=1)


def _wspecs():
  wspec = lambda l, a, b: pl.BlockSpec((1, a, b), lambda i: (l, 0, 0))
  bspec = lambda a: pl.BlockSpec((2, a), lambda i: (0, 0))
  return lambda l: [wspec(l, NDIM, NDIM), wspec(l, 3 * HD, NDIM),
                    wspec(l, 3 * HD, HD), bspec(NDIM), bspec(3 * HD),
                    bspec(3 * HD)]


def _tc_layer0():
  data = lambda cols: pl.BlockSpec((BN, cols), lambda i: (i, 0))
  gview = lambda d: pl.BlockSpec((1, BN, D0), lambda i: (d, i, 0))
  w = _wspecs()(0)
  return pl.pallas_call(
      _tc_body0,
      grid=(N // BN,),
      in_specs=[data(NDIM), gview(0), gview(1)] + w + w,
      out_specs=[data(NDIM), data(NDIM), data(HD), data(HD)],
      out_shape=[jax.ShapeDtypeStruct((N, NDIM), jnp.float32)] * 2 + [
          jax.ShapeDtypeStruct((N, HD), jnp.float32)] * 2,
  )


def _tc_layer1():
  data = lambda cols: pl.BlockSpec((BN, cols), lambda i: (i, 0))
  gview = lambda d: pl.BlockSpec((1, BN, HD), lambda i: (d, i, 0))
  w = _wspecs()(1)
  return pl.pallas_call(
      _tc_body1,
      grid=(N // BN,),
      in_specs=[data(NDIM), data(NDIM), gview(0), gview(1)] + w + w,
      out_specs=[data(NDIM)],
      out_shape=[jax.ShapeDtypeStruct((N, NDIM), jnp.float32)],
  )


def kernel(h, edge_index, Wmsg_f, bmsg_f, Wih_f, Whh_f, bih_f, bhh_f,
           Wmsg_b, bmsg_b, Wih_b, Whh_b, bih_b, bhh_b):
  em = edge_index.reshape(2, E // CH, CH)

  rnd = lambda x: x.astype(jnp.bfloat16).astype(jnp.float32)
  pad = jnp.concatenate(
      [jnp.ones((N, 1), jnp.float32),
       jnp.zeros((N, D0 - HD - 1), jnp.float32)], axis=1)
  tf0 = jnp.concatenate([rnd(h[:, :HD]), pad], axis=1)
  tb0 = jnp.concatenate([rnd(h[:, HD:]), pad], axis=1)

  g0 = _sc_scatter(D0)(tf0, tb0, em, jnp.zeros((ROWS_A, D0), jnp.float32))
  wf = (Wmsg_f, Wih_f, Whh_f, bmsg_f, bih_f, bhh_f)
  wb = (Wmsg_b, Wih_b, Whh_b, bmsg_b, bih_b, bhh_b)
  hdf, hdb, tf1, tb1 = _tc_layer0()(h, g0, g0, *wf, *wb)

  g1 = _sc_scatter(HD)(tf1, tb1, em, jnp.zeros((ROWS_A, HD), jnp.float32))
  (out,) = _tc_layer1()(hdf, hdb, g1, g1, *wf, *wb)
  return out


# speedup vs baseline: 1.1483x; 1.0335x over previous
import functools

import jax
import jax.numpy as jnp
from jax import lax
from jax.experimental import pallas as pl
from jax.experimental.pallas import tpu as pltpu
from jax.experimental.pallas import tpu_sc as plsc

N = 10000
HD = 64
E = 320000
NDIM = 128

NSUB = 16
CH = 125
EPT = E // NSUB
CPT = EPT // CH
ROWS_A = 640
ROWS_B = N - 15 * ROWS_A
D0 = 72
NB = 4


def _sc_scatter(D):
  mesh = plsc.VectorSubcoreMesh(core_axis_name="c", subcore_axis_name="s")

  def body(tf, tb, em, zrows, g_out, idxg, idxs, rows, g_sh, semg, sems):
    c = lax.axis_index("c")
    s = lax.axis_index("s")

    @pl.when(c == 0)
    def _():
      pltpu.sync_copy(em.at[0, pl.ds(s * CPT, CPT)], idxg)
      pltpu.sync_copy(em.at[1, pl.ds(s * CPT, CPT)], idxs)

    @pl.when(c == 1)
    def _():
      pltpu.sync_copy(em.at[1, pl.ds(s * CPT, CPT)], idxg)
      pltpu.sync_copy(em.at[0, pl.ds(s * CPT, CPT)], idxs)

    @pl.when(s < 15)
    def _():
      pltpu.sync_copy(zrows, g_sh.at[pl.ds(s * ROWS_A, ROWS_A)])

    @pl.when(s == 15)
    def _():
      pltpu.sync_copy(zrows.at[pl.ds(0, ROWS_B)],
                      g_sh.at[pl.ds(15 * ROWS_A, ROWS_B)])

    plsc.subcore_barrier()

    def run_dir(table, cidx):
      for b in range(NB):
        pltpu.async_copy(table.at[idxg.at[b]], rows.at[b], semg)

      def group(i, carry):
        for b in range(NB):
          j = i * NB + b
          pltpu.make_async_copy(table.at[idxg.at[j]], rows.at[b], semg).wait()
          pltpu.async_copy(rows.at[b], g_sh.at[idxs.at[j]], sems, add=True)
          pltpu.make_async_copy(rows.at[b], g_sh.at[idxs.at[j]], sems).wait()

          @pl.when(j < CPT - NB)
          def _():
            pltpu.async_copy(table.at[idxg.at[j + NB]], rows.at[b], semg)

        return carry

      lax.fori_loop(0, CPT // NB, group, 0)
      plsc.subcore_barrier()

      @pl.when(s < 15)
      def _():
        pltpu.sync_copy(g_sh.at[pl.ds(s * ROWS_A, ROWS_A)],
                        g_out.at[cidx, pl.ds(s * ROWS_A, ROWS_A)])

      @pl.when(s == 15)
      def _():
        pltpu.sync_copy(g_sh.at[pl.ds(15 * ROWS_A, ROWS_B)],
                        g_out.at[cidx, pl.ds(15 * ROWS_A, ROWS_B)])

    @pl.when(c == 0)
    def _():
      run_dir(tf, 0)

    @pl.when(c == 1)
    def _():
      run_dir(tb, 1)

  return pl.kernel(
      body,
      out_type=jax.ShapeDtypeStruct((2, N, D), jnp.float32),
      mesh=mesh,
      compiler_params=pltpu.CompilerParams(use_tc_tiling_on_sc=False),
      scratch_types=[
          pltpu.VMEM((CPT, CH), jnp.int32),
          pltpu.VMEM((CPT, CH), jnp.int32),
          pltpu.VMEM((NB, CH, D), jnp.float32),
          pltpu.VMEM_SHARED((N, D), jnp.float32),
          pltpu.SemaphoreType.DMA,
          pltpu.SemaphoreType.DMA,
      ],
  )


BN = 2000


def _gru_dir(l, h, g, deg, wm, wih, whh, bm, bih, bhh):
  f32 = jnp.float32
  bf = jnp.bfloat16
  wm_ = wm[...].reshape(NDIM, NDIM)
  w1r = wm_[:, :HD].astype(bf).astype(f32)
  w2 = wm_[:, HD:].astype(bf)
  h16 = h.astype(bf)
  ddot = lambda a, b, **kw: lax.dot_general(
      a, b, (((1,), (1,)), ((), ())), preferred_element_type=f32, **kw)
  b_part = ddot(h16, w2) + bm[...][l:l + 1]
  aggr = ddot(g, w1r, precision=lax.Precision.HIGHEST) + deg * b_part
  gi = ddot(aggr.astype(bf), wih[...].reshape(3 * HD, NDIM).astype(bf))
  gi = gi + bih[...][l:l + 1]
  gh = ddot(h16, whh[...].reshape(3 * HD, HD).astype(bf)) + bhh[...][l:l + 1]
  r = jax.nn.sigmoid(gi[:, :HD] + gh[:, :HD])
  z = jax.nn.sigmoid(gi[:, HD:2 * HD] + gh[:, HD:2 * HD])
  n = jnp.tanh(gi[:, 2 * HD:] + r * gh[:, 2 * HD:])
  return (1.0 - z) * n + z * h


def _tc_body0(hin, gf, gb,
              wmf, wihf, whhf, bmf, bihf, bhhf,
              wmb, wihb, whhb, bmb, bihb, bhhb,
              hdf, hdb, tf1, tb1):
  h128 = hin[...]
  rnd16 = lambda x: x.astype(jnp.bfloat16).astype(jnp.float32)
  zpad = jnp.zeros((h128.shape[0], NDIM - HD - 1), jnp.float32)
  for (gref, c0, hd_out, t_out, w) in (
      (gf, 0, hdf, tf1, (wmf, wihf, whhf, bmf, bihf, bhhf)),
      (gb, HD, hdb, tb1, (wmb, wihb, whhb, bmb, bihb, bhhb))):
    g80 = gref[...].reshape(-1, D0)
    h2 = _gru_dir(0, h128[:, c0:c0 + HD], g80[:, :HD], g80[:, HD:HD + 1], *w)
    hd_out[...] = jnp.concatenate([h2, g80[:, HD:HD + 1], zpad], axis=1)
    t_out[...] = rnd16(h2)


def _tc_body1(hdf, hdb, gf, gb,
              wmf, wihf, whhf, bmf, bihf, bhhf,
              wmb, wihb, whhb, bmb, bihb, bhhb,
              out):
  res = []
  for (hd, gref, w) in (
      (hdf, gf, (wmf, wihf, whhf, bmf, bihf, bhhf)),
      (hdb, gb, (wmb, wihb, whhb, bmb, bihb, bhhb))):
    h128 = hd[...]
    g = gref[...].reshape(-1, HD)
    res.append(_gru_dir(1, h128[:, :HD], g, h128[:, HD:HD + 1], *w))
  out[...] = jnp.concatenate(res, axis=1)


def _wspecs():
  wspec = lambda l, a, b: pl.BlockSpec((1, a, b), lambda i: (l, 0, 0))
  bspec = lambda a: pl.BlockSpec((2, a), lambda i: (0, 0))
  return lambda l: [wspec(l, NDIM, NDIM), wspec(l, 3 * HD, NDIM),
                    wspec(l, 3 * HD, HD), bspec(NDIM), bspec(3 * HD),
                    bspec(3 * HD)]


def _tc_layer0():
  data = lambda cols: pl.BlockSpec((BN, cols), lambda i: (i, 0))
  gview = lambda d: pl.BlockSpec((1, BN, D0), lambda i: (d, i, 0))
  w = _wspecs()(0)
  return pl.pallas_call(
      _tc_body0,
      grid=(N // BN,),
      in_specs=[data(NDIM), gview(0), gview(1)] + w + w,
      out_specs=[data(NDIM), data(NDIM), data(HD), data(HD)],
      out_shape=[jax.ShapeDtypeStruct((N, NDIM), jnp.float32)] * 2 + [
          jax.ShapeDtypeStruct((N, HD), jnp.float32)] * 2,
  )


def _tc_layer1():
  data = lambda cols: pl.BlockSpec((BN, cols), lambda i: (i, 0))
  gview = lambda d: pl.BlockSpec((1, BN, HD), lambda i: (d, i, 0))
  w = _wspecs()(1)
  return pl.pallas_call(
      _tc_body1,
      grid=(N // BN,),
      in_specs=[data(NDIM), data(NDIM), gview(0), gview(1)] + w + w,
      out_specs=[data(NDIM)],
      out_shape=[jax.ShapeDtypeStruct((N, NDIM), jnp.float32)],
  )


def kernel(h, edge_index, Wmsg_f, bmsg_f, Wih_f, Whh_f, bih_f, bhh_f,
           Wmsg_b, bmsg_b, Wih_b, Whh_b, bih_b, bhh_b):
  em = edge_index.reshape(2, E // CH, CH)

  rnd = lambda x: x.astype(jnp.bfloat16).astype(jnp.float32)
  pad = jnp.concatenate(
      [jnp.ones((N, 1), jnp.float32),
       jnp.zeros((N, D0 - HD - 1), jnp.float32)], axis=1)
  tf0 = jnp.concatenate([rnd(h[:, :HD]), pad], axis=1)
  tb0 = jnp.concatenate([rnd(h[:, HD:]), pad], axis=1)

  g0 = _sc_scatter(D0)(tf0, tb0, em, jnp.zeros((ROWS_A, D0), jnp.float32))
  wf = (Wmsg_f, Wih_f, Whh_f, bmsg_f, bih_f, bhh_f)
  wb = (Wmsg_b, Wih_b, Whh_b, bmsg_b, bih_b, bhh_b)
  hdf, hdb, tf1, tb1 = _tc_layer0()(h, g0, g0, *wf, *wb)

  g1 = _sc_scatter(HD)(tf1, tb1, em, jnp.zeros((ROWS_A, HD), jnp.float32))
  (out,) = _tc_layer1()(hdf, hdb, g1, g1, *wf, *wb)
  return out
